# baseline (speedup 1.0000x reference)
import jax
import jax.numpy as jnp
from jax.experimental import pallas as pl
from jax.experimental.pallas import tpu as pltpu

_TAPS = [(i, j) for i in range(3) for j in range(3)]
_NB = 32


def _net_kernel(xin_ref, wct_ref, mask_ref, ind_ref, out_ref, rhs_ref, xk_ref):
    @pl.when(pl.program_id(0) == 0)
    def _init_constant_rows():
        xk_ref[...] = jnp.zeros((4, 144, 4096), jnp.bfloat16)
        for rb in range(4):
            for t in range(9):
                xk_ref[rb, 16 * t + 9, :] = ind_ref[t]
            xk_ref[rb, 10, :] = ind_ref[9]

    for b in range(_NB):
        for t, (di, dj) in enumerate(_TAPS):
            s = 64 * di + dj
            rhs_ref[t, b * 4352:b * 4352 + 4352] = (
                xin_ref[b, s:s + 4352] * mask_ref[0])
    for b in range(_NB):
        rb = b % 4
        for t, (ei, ej) in enumerate(_TAPS):
            s = b * 4352 + 64 * ei + ej
            xk_ref[rb, 16 * t:16 * t + 9, :] = rhs_ref[:, s:s + 4096]
        out_ref[b] = jnp.dot(wct_ref[:], xk_ref[rb],
                             preferred_element_type=jnp.float32)


def kernel(x, W1, b1, W2, b2):
    n = x.shape[0]
    xin = jnp.pad(x.reshape(n, 4096), ((0, 0), (130, 382))).astype(jnp.bfloat16)

    w2r = W2.reshape(9, 64, 32)
    wc = jnp.einsum('uc,tco->otu', W1.reshape(9, 64), w2r)
    bt = jnp.einsum('c,tco->to', b1, w2r)
    wct = jnp.zeros((32, 9, 16), jnp.float32)
    wct = wct.at[:, :, :9].set(wc)
    wct = wct.at[:, :, 9].set(bt.T)
    wct = wct.at[:, 0, 10].set(b2)
    wct = wct.reshape(32, 144).astype(jnp.bfloat16)

    ll = jnp.arange(4352)
    mask = ((ll >= 130) & (ll < 4098) & ((ll - 130) % 64 < 62))
    mask = mask.astype(jnp.bfloat16)[None, :]

    pq = jnp.arange(64)
    inds = []
    for t, (ei, ej) in enumerate(_TAPS):
        rowok = (pq + ei - 2 >= 0) & (pq + ei - 2 <= 61)
        colok = (pq + ej - 2 >= 0) & (pq + ej - 2 <= 61)
        inds.append((rowok[:, None] & colok[None, :]).reshape(4096))
    inds.append(jnp.ones((4096,), jnp.bool_))
    ind = jnp.stack(inds).astype(jnp.bfloat16)

    out = pl.pallas_call(
        _net_kernel,
        grid=(n // _NB,),
        in_specs=[
            pl.BlockSpec((_NB, 4608), lambda i: (i, 0)),
            pl.BlockSpec((32, 144), lambda i: (0, 0)),
            pl.BlockSpec((1, 4352), lambda i: (0, 0)),
            pl.BlockSpec((10, 4096), lambda i: (0, 0)),
        ],
        out_specs=pl.BlockSpec((_NB, 32, 4096), lambda i: (i, 0, 0)),
        out_shape=jax.ShapeDtypeStruct((n, 32, 4096), jnp.float32),
        scratch_shapes=[pltpu.VMEM((9, _NB * 4352), jnp.bfloat16),
                        pltpu.VMEM((4, 144, 4096), jnp.bfloat16)],
    )(xin, wct, mask, ind)
    return out.reshape(n, 32, 64, 64)


# speedup vs baseline: 1.0114x; 1.0114x over previous
import jax
import jax.numpy as jnp
from jax.experimental import pallas as pl
from jax.experimental.pallas import tpu as pltpu

_TAPS = [(i, j) for i in range(3) for j in range(3)]
_NB = 32


def _net_kernel(xin_ref, wct_ref, mask_ref, bias_ref, out_ref, rhs_ref, xk_ref):
    @pl.when(pl.program_id(0) == 0)
    def _zero_junk_rows():
        xk_ref[...] = jnp.zeros((4, 144, 4096), jnp.bfloat16)

    for b in range(_NB):
        for t, (di, dj) in enumerate(_TAPS):
            s = 64 * di + dj
            rhs_ref[t, b * 4352:b * 4352 + 4352] = (
                xin_ref[b, s:s + 4352] * mask_ref[0])
    for b in range(_NB):
        rb = b % 4
        for t, (ei, ej) in enumerate(_TAPS):
            s = b * 4352 + 64 * ei + ej
            xk_ref[rb, 16 * t:16 * t + 9, :] = rhs_ref[:, s:s + 4096]
        z = bias_ref[:] + jnp.dot(wct_ref[:], xk_ref[rb],
                                  preferred_element_type=jnp.float32)
        out_ref[b] = z


def kernel(x, W1, b1, W2, b2):
    n = x.shape[0]
    xin = jnp.pad(x.reshape(n, 4096), ((0, 0), (130, 382))).astype(jnp.bfloat16)

    w2r = W2.reshape(9, 64, 32)
    wc = jnp.einsum('uc,tco->otu', W1.reshape(9, 64), w2r)
    wct = jnp.zeros((32, 9, 16), jnp.float32)
    wct = wct.at[:, :, :9].set(wc).reshape(32, 144).astype(jnp.bfloat16)

    ll = jnp.arange(4352)
    mask = ((ll >= 130) & (ll < 4098) & ((ll - 130) % 64 < 62))
    mask = mask.astype(jnp.bfloat16)[None, :]

    bt = jnp.einsum('c,tco->to', b1, w2r)
    pq = jnp.arange(64)
    plane = jnp.zeros((32, 64, 64), jnp.float32) + b2[:, None, None]
    for t, (ei, ej) in enumerate(_TAPS):
        rowok = (pq + ei - 2 >= 0) & (pq + ei - 2 <= 61)
        colok = (pq + ej - 2 >= 0) & (pq + ej - 2 <= 61)
        m = (rowok[:, None] & colok[None, :]).astype(jnp.float32)
        plane = plane + bt[t][:, None, None] * m[None, :, :]
    bias = plane.reshape(32, 4096)

    out = pl.pallas_call(
        _net_kernel,
        grid=(n // _NB,),
        in_specs=[
            pl.BlockSpec((_NB, 4608), lambda i: (i, 0)),
            pl.BlockSpec((32, 144), lambda i: (0, 0)),
            pl.BlockSpec((1, 4352), lambda i: (0, 0)),
            pl.BlockSpec((32, 4096), lambda i: (0, 0)),
        ],
        out_specs=pl.BlockSpec((_NB, 32, 4096), lambda i: (i, 0, 0)),
        out_shape=jax.ShapeDtypeStruct((n, 32, 4096), jnp.float32),
        scratch_shapes=[pltpu.VMEM((9, _NB * 4352), jnp.bfloat16),
                        pltpu.VMEM((4, 144, 4096), jnp.bfloat16)],
    )(xin, wct, mask, bias)
    return out.reshape(n, 32, 64, 64)
